# Initial kernel scaffold; baseline (speedup 1.0000x reference)
import jax
import jax.numpy as jnp
from jax.experimental import pallas as pl


def kernel(transformation, sig):
    raise NotImplementedError("write your pallas kernel here")


# speedup vs baseline: 4.8618x; 4.8618x over previous
import functools

import jax
import jax.numpy as jnp
from jax import lax
from jax.experimental import pallas as pl
from jax.experimental.pallas import tpu as pltpu
from jax.experimental.pallas import tpu_sc as plsc

_OUT_LEN = 4096
_IN_LEN = 8192
_B = 32
_C = 32
_NPAIR = _B * _C

_INFO = plsc.get_sparse_core_info()
_NC = _INFO.num_cores
_NS = _INFO.num_subcores
_NW = _NC * _NS
_ROWS_PER_W = _NPAIR // _NW
_NBLK = _OUT_LEN // 16


@functools.partial(
    pl.kernel,
    out_type=jax.ShapeDtypeStruct((_NPAIR, _OUT_LEN), jnp.float32),
    mesh=plsc.VectorSubcoreMesh(core_axis_name="c", subcore_axis_name="s"),
    scratch_types=[
        pltpu.VMEM((_IN_LEN,), jnp.float32),
        pltpu.VMEM((_OUT_LEN,), jnp.float32),
        pltpu.VMEM((_OUT_LEN,), jnp.float32),
        pltpu.VMEM((128,), jnp.float32),
    ],
    compiler_params=pltpu.CompilerParams(needs_layout_passes=False),
)
def _sc_interp(sigT_hbm, traf_hbm, lin_hbm, out_hbm, row_v, orow_v, lin_v, traf_v):
    wid = lax.axis_index("s") * _NC + lax.axis_index("c")
    base = wid * _ROWS_PER_W
    pltpu.sync_copy(lin_hbm, lin_v)
    pltpu.sync_copy(
        traf_hbm.at[pl.ds(base * 2, _ROWS_PER_W * 2)],
        traf_v.at[pl.ds(0, _ROWS_PER_W * 2)],
    )

    def row_body(j, carry):
        p = base + j
        pltpu.sync_copy(sigT_hbm.at[p], row_v)
        a0 = plsc.load_gather(traf_v, [jnp.full((16,), 2 * j, jnp.int32)])
        a1 = plsc.load_gather(traf_v, [jnp.full((16,), 2 * j + 1, jnp.int32)])

        def blk_body(blk, c2):
            o = blk * 16
            linv = lin_v[pl.ds(o, 16)]
            x = (a0 * linv + a1) * jnp.float32(4096.0)
            x0 = jnp.maximum(jnp.minimum(x.astype(jnp.int32), _IN_LEN - 1), 0)
            x1 = jnp.minimum(x0 + 1, _IN_LEN - 1)
            v0 = plsc.load_gather(row_v, [x0])
            v1 = plsc.load_gather(row_v, [x1])
            w0 = x1.astype(jnp.float32) - x
            w1 = x - x0.astype(jnp.float32)
            orow_v[pl.ds(o, 16)] = w0 * v0 + w1 * v1
            return c2

        lax.fori_loop(0, _NBLK, blk_body, 0, unroll=4)
        pltpu.sync_copy(orow_v, out_hbm.at[p])
        return carry

    lax.fori_loop(0, _ROWS_PER_W, row_body, 0)


def kernel(transformation, sig):
    sigT = jnp.transpose(sig, (0, 2, 1)).reshape(_NPAIR, _IN_LEN)
    traf = lax.reduce_precision(
        transformation.reshape(_NPAIR * 2), exponent_bits=8, mantissa_bits=7
    )
    lin = lax.reduce_precision(
        jnp.linspace(0.0, 1.0, _OUT_LEN), exponent_bits=8, mantissa_bits=7
    )
    outT = _sc_interp(sigT, traf, lin)
    return jnp.transpose(outT.reshape(_B, _C, _OUT_LEN), (0, 2, 1))


# speedup vs baseline: 17.9249x; 3.6869x over previous
import functools

import jax
import jax.numpy as jnp
from jax import lax
from jax.experimental import pallas as pl
from jax.experimental.pallas import tpu as pltpu
from jax.experimental.pallas import tpu_sc as plsc

_OUT_LEN = 4096
_IN_LEN = 8192
_B = 32
_C = 32
_NPAIR = _B * _C

_INFO = plsc.get_sparse_core_info()
_NC = _INFO.num_cores
_NS = _INFO.num_subcores
_NW = _NC * _NS
_ROWS_PER_W = _NPAIR // _NW


@functools.partial(
    pl.kernel,
    out_type=jax.ShapeDtypeStruct((_NPAIR, _OUT_LEN), jnp.float32),
    mesh=plsc.VectorSubcoreMesh(core_axis_name="c", subcore_axis_name="s"),
    scratch_types=[
        pltpu.VMEM((_IN_LEN,), jnp.float32),
        pltpu.VMEM((_IN_LEN,), jnp.float32),
        pltpu.VMEM((_OUT_LEN,), jnp.float32),
        pltpu.VMEM((_OUT_LEN,), jnp.float32),
        pltpu.VMEM((_OUT_LEN,), jnp.float32),
        pltpu.VMEM((128,), jnp.float32),
        pltpu.SemaphoreType.DMA,
        pltpu.SemaphoreType.DMA,
        pltpu.SemaphoreType.DMA,
        pltpu.SemaphoreType.DMA,
    ],
    compiler_params=pltpu.CompilerParams(needs_layout_passes=False),
)
def _sc_interp(
    sigT_hbm, traf_hbm, lin_hbm, out_hbm,
    row0_v, row1_v, orow0_v, orow1_v, lin_v, traf_v,
    si0, si1, so0, so1,
):
    wid = lax.axis_index("s") * _NC + lax.axis_index("c")
    base = wid * _ROWS_PER_W
    pltpu.sync_copy(lin_hbm, lin_v)
    pltpu.sync_copy(
        traf_hbm.at[pl.ds(base * 2, _ROWS_PER_W * 2)],
        traf_v.at[pl.ds(0, _ROWS_PER_W * 2)],
    )
    pltpu.async_copy(sigT_hbm.at[base], row0_v, si0)

    def do_row(p, j, row_v, orow_v, so):
        a0 = plsc.load_gather(traf_v, [jnp.full((16,), 2 * j, jnp.int32)])
        a1 = plsc.load_gather(traf_v, [jnp.full((16,), 2 * j + 1, jnp.int32)])
        scale_a = a0 * jnp.float32(4096.0)
        scale_b = a1 * jnp.float32(4096.0)

        @plsc.parallel_loop(0, _OUT_LEN, step=16, unroll=8)
        def blk(i):
            linv = lin_v[pl.ds(i, 16)]
            x = scale_a * linv + scale_b
            x0 = jnp.minimum(x.astype(jnp.int32), _IN_LEN - 1)
            x1 = jnp.minimum(x0 + 1, _IN_LEN - 1)
            v0 = plsc.load_gather(row_v, [x0])
            v1 = plsc.load_gather(row_v, [x1])
            w0 = x1.astype(jnp.float32) - x
            w1 = x - x0.astype(jnp.float32)
            orow_v[pl.ds(i, 16)] = w0 * v0 + w1 * v1

        pltpu.async_copy(orow_v, out_hbm.at[p], so)

    def pair_body(k, carry):
        p0 = base + 2 * k
        pltpu.make_async_copy(sigT_hbm.at[p0], row0_v, si0).wait()
        pltpu.async_copy(sigT_hbm.at[p0 + 1], row1_v, si1)

        @pl.when(k > 0)
        def _():
            pltpu.make_async_copy(orow0_v, out_hbm.at[p0 - 2], so0).wait()

        do_row(p0, 2 * k, row0_v, orow0_v, so0)

        pltpu.make_async_copy(sigT_hbm.at[p0 + 1], row1_v, si1).wait()

        @pl.when(k < _ROWS_PER_W // 2 - 1)
        def _():
            pltpu.async_copy(sigT_hbm.at[p0 + 2], row0_v, si0)

        @pl.when(k > 0)
        def _():
            pltpu.make_async_copy(orow1_v, out_hbm.at[p0 - 1], so1).wait()

        do_row(p0 + 1, 2 * k + 1, row1_v, orow1_v, so1)
        return carry

    lax.fori_loop(0, _ROWS_PER_W // 2, pair_body, 0)
    pltpu.make_async_copy(orow0_v, out_hbm.at[base + _ROWS_PER_W - 2], so0).wait()
    pltpu.make_async_copy(orow1_v, out_hbm.at[base + _ROWS_PER_W - 1], so1).wait()


def kernel(transformation, sig):
    sigT = jnp.transpose(sig, (0, 2, 1)).reshape(_NPAIR, _IN_LEN)
    traf = lax.reduce_precision(
        transformation.reshape(_NPAIR * 2), exponent_bits=8, mantissa_bits=7
    )
    lin = lax.reduce_precision(
        jnp.linspace(0.0, 1.0, _OUT_LEN), exponent_bits=8, mantissa_bits=7
    )
    outT = _sc_interp(sigT, traf, lin)
    return jnp.transpose(outT.reshape(_B, _C, _OUT_LEN), (0, 2, 1))
